# baseline (speedup 1.0000x reference)
import functools

import jax
import jax.numpy as jnp
from jax import lax
from jax.experimental import pallas as pl
from jax.experimental.pallas import tpu as pltpu
from jax.experimental.pallas import tpu_sc as plsc

N = 10000
D = 128
E_TOT = 330000
NW = 32
K = 64
NB = 162
EW = NB * K
EPAD = NW * EW
NP = NB // 2
DROWS = 80

_f32 = jnp.float32


def _sc_gat_body(h_hbm, asrc_hbm, adst_hbm, srcf_hbm, dstf_hbm,
                 acc_out, den_out,
                 src_f, dst_f, denom_l, buf0, buf1, asg0, adg0, asg1, adg1,
                 acc_sh, den_sh, asrc_sp, adst_sp,
                 semr0, semr1, sema0, sema1, sems0, sems1):
    c = lax.axis_index("c")
    s = lax.axis_index("s")
    wid = s * 2 + c
    iota16 = lax.iota(jnp.int32, 16)
    zeros16 = jnp.zeros((16,), _f32)

    pltpu.sync_copy(srcf_hbm.at[wid], src_f)
    pltpu.sync_copy(dstf_hbm.at[wid], dst_f)

    @pl.when(s == 0)
    def _():
        pltpu.sync_copy(asrc_hbm, asrc_sp)
        pltpu.sync_copy(adst_hbm, adst_sp)

    for r in range(DROWS):
        for cc in range(8):
            denom_l[r, pl.ds(16 * cc, 16)] = zeros16
    for r in range(K):
        for cc in range(8):
            buf0[r, pl.ds(16 * cc, 16)] = zeros16

    abase = pl.multiple_of(s * 632, 8)

    @pl.when(s < 15)
    def _():
        for r in range(9):
            pltpu.sync_copy(buf0, acc_sh.at[pl.ds(abase + 64 * r, 64)])
        pltpu.sync_copy(buf0.at[pl.ds(0, 56)], acc_sh.at[pl.ds(abase + 576, 56)])

    @pl.when(s == 15)
    def _():
        for r in range(8):
            pltpu.sync_copy(buf0, acc_sh.at[pl.ds(abase + 64 * r, 64)])
        pltpu.sync_copy(buf0.at[pl.ds(0, 8)], acc_sh.at[pl.ds(abase + 512, 8)])

    dbase = pl.multiple_of(s * 8, 8)

    @pl.when(s < 10)
    def _():
        pltpu.sync_copy(denom_l.at[pl.ds(0, 8)], den_sh.at[pl.ds(dbase, 8)])

    plsc.subcore_barrier()

    def copies(j, buf, asg, adg, semr, sema):
        sidx = src_f.at[pl.ds(j * K, K)]
        didx = dst_f.at[pl.ds(j * K, K)]
        return (pltpu.make_async_copy(h_hbm.at[sidx], buf, semr),
                pltpu.make_async_copy(asrc_sp.at[sidx], asg, sema),
                pltpu.make_async_copy(adst_sp.at[didx], adg, sema))

    def fire(j, buf, asg, adg, semr, sema):
        for cp in copies(j, buf, asg, adg, semr, sema):
            cp.start()

    def wait(j, buf, asg, adg, semr, sema):
        for cp in copies(j, buf, asg, adg, semr, sema):
            cp.wait()

    def compute(j, buf, asg, adg, sems):
        base = wid * EW + j * K
        dis = []
        for sg in range(4):
            sl16 = pl.ds(16 * sg, 16)
            di = dst_f[pl.ds(j * K + 16 * sg, 16)]
            e = asg[sl16] + adg[sl16]
            e = jnp.where(e >= 0.0, e, e * jnp.float32(0.2))
            ex = jnp.exp(e)
            ex = jnp.where(base + 16 * sg + iota16 < E_TOT, ex, 0.0)
            plsc.addupdate_scatter(denom_l, [di >> 7, di & 127], ex)
            for k in range(16):
                bc = jnp.full((16,), ex[k], _f32)
                row = 16 * sg + k
                for cc in range(8):
                    cs = pl.ds(16 * cc, 16)
                    buf[row, cs] = buf[row, cs] * bc
            dis.append(di)
        for sg in range(4):
            pltpu.make_async_copy(buf.at[pl.ds(16 * sg, 16)],
                                  acc_sh.at[dis[sg]], sems).start(add=True)

    def wait_scat(buf, sems):
        for sg in range(4):
            pltpu.make_async_copy(buf.at[pl.ds(16 * sg, 16)],
                                  acc_sh.at[iota16], sems).wait()

    fire(0, buf0, asg0, adg0, semr0, sema0)

    def body(jp, carry):
        j0 = 2 * jp
        j1 = j0 + 1

        @pl.when(jp > 0)
        def _():
            wait_scat(buf1, sems1)

        fire(j1, buf1, asg1, adg1, semr1, sema1)
        wait(j0, buf0, asg0, adg0, semr0, sema0)
        compute(j0, buf0, asg0, adg0, sems0)

        @pl.when(jp < NP - 1)
        def _():
            wait_scat(buf0, sems0)
            fire(j0 + 2, buf0, asg0, adg0, semr0, sema0)

        wait(j1, buf1, asg1, adg1, semr1, sema1)
        compute(j1, buf1, asg1, adg1, sems1)
        return carry

    lax.fori_loop(0, NP, body, 0)
    wait_scat(buf0, sems0)
    wait_scat(buf1, sems1)

    for g in range(5):
        pltpu.sync_copy(denom_l.at[pl.ds(16 * g, 16)],
                        den_sh.at[iota16 + 16 * g], add=True)
    plsc.subcore_barrier()

    @pl.when(s < 15)
    def _():
        pltpu.sync_copy(acc_sh.at[pl.ds(abase, 632)],
                        acc_out.at[c, pl.ds(abase, 632)])

    @pl.when(s == 15)
    def _():
        pltpu.sync_copy(acc_sh.at[pl.ds(abase, 520)],
                        acc_out.at[c, pl.ds(abase, 520)])

    @pl.when(s < 10)
    def _():
        pltpu.sync_copy(den_sh.at[pl.ds(dbase, 8)],
                        den_out.at[c, pl.ds(dbase, 8)])


_sc_gat = functools.partial(
    pl.kernel,
    out_type=(jax.ShapeDtypeStruct((2, N, D), _f32),
              jax.ShapeDtypeStruct((2, DROWS, D), _f32)),
    mesh=plsc.VectorSubcoreMesh(core_axis_name="c", subcore_axis_name="s"),
    compiler_params=pltpu.CompilerParams(needs_layout_passes=False),
    scratch_types=[
        pltpu.VMEM((EW,), jnp.int32),
        pltpu.VMEM((EW,), jnp.int32),
        pltpu.VMEM((DROWS, D), _f32),
        pltpu.VMEM((K, D), _f32),
        pltpu.VMEM((K, D), _f32),
        pltpu.VMEM((K,), _f32),
        pltpu.VMEM((K,), _f32),
        pltpu.VMEM((K,), _f32),
        pltpu.VMEM((K,), _f32),
        pltpu.VMEM_SHARED((N, D), _f32),
        pltpu.VMEM_SHARED((DROWS, D), _f32),
        pltpu.VMEM_SHARED((N,), _f32),
        pltpu.VMEM_SHARED((N,), _f32),
        pltpu.SemaphoreType.DMA,
        pltpu.SemaphoreType.DMA,
        pltpu.SemaphoreType.DMA,
        pltpu.SemaphoreType.DMA,
        pltpu.SemaphoreType.DMA,
        pltpu.SemaphoreType.DMA,
    ],
)(_sc_gat_body)


def _proj_body(x_ref, w_ref, as_ref, ad_ref, h_ref, asrc_ref, adst_ref):
    h = jnp.dot(x_ref[...], w_ref[...], preferred_element_type=_f32)
    h_ref[...] = h
    asrc_ref[...] = jnp.dot(h, as_ref[...], preferred_element_type=_f32)
    adst_ref[...] = jnp.dot(h, ad_ref[...], preferred_element_type=_f32)


def _tc_proj(x, w, a_s, a_d):
    return pl.pallas_call(
        _proj_body,
        grid=(10,),
        in_specs=[pl.BlockSpec((1000, D), lambda i: (i, 0)),
                  pl.BlockSpec((D, D), lambda i: (0, 0)),
                  pl.BlockSpec((D, 1), lambda i: (0, 0)),
                  pl.BlockSpec((D, 1), lambda i: (0, 0))],
        out_specs=[pl.BlockSpec((1000, D), lambda i: (i, 0)),
                   pl.BlockSpec((1000, 1), lambda i: (i, 0)),
                   pl.BlockSpec((1000, 1), lambda i: (i, 0))],
        out_shape=[jax.ShapeDtypeStruct((N, D), _f32),
                   jax.ShapeDtypeStruct((N, 1), _f32),
                   jax.ShapeDtypeStruct((N, 1), _f32)],
    )(x, w, a_s, a_d)


def _comb_proj_body(acc_ref, den_ref, b_ref, w_ref, as_ref, ad_ref,
                    e_ref, h_ref, asrc_ref, adst_ref):
    r = 1.0 / (den_ref[:, 0:1] + den_ref[:, 1:2] + 1e-16)
    e = (acc_ref[0] + acc_ref[1]) * r + b_ref[...]
    e_ref[...] = e
    h = jnp.dot(e, w_ref[...], preferred_element_type=_f32)
    h_ref[...] = h
    asrc_ref[...] = jnp.dot(h, as_ref[...], preferred_element_type=_f32)
    adst_ref[...] = jnp.dot(h, ad_ref[...], preferred_element_type=_f32)


def _tc_comb_proj(acc, denp, b, w, a_s, a_d):
    return pl.pallas_call(
        _comb_proj_body,
        grid=(10,),
        in_specs=[pl.BlockSpec((2, 1000, D), lambda i: (0, i, 0)),
                  pl.BlockSpec((1000, D), lambda i: (i, 0)),
                  pl.BlockSpec((1, D), lambda i: (0, 0)),
                  pl.BlockSpec((D, D), lambda i: (0, 0)),
                  pl.BlockSpec((D, 1), lambda i: (0, 0)),
                  pl.BlockSpec((D, 1), lambda i: (0, 0))],
        out_specs=[pl.BlockSpec((1000, D), lambda i: (i, 0)),
                   pl.BlockSpec((1000, D), lambda i: (i, 0)),
                   pl.BlockSpec((1000, 1), lambda i: (i, 0)),
                   pl.BlockSpec((1000, 1), lambda i: (i, 0))],
        out_shape=[jax.ShapeDtypeStruct((N, D), _f32),
                   jax.ShapeDtypeStruct((N, D), _f32),
                   jax.ShapeDtypeStruct((N, 1), _f32),
                   jax.ShapeDtypeStruct((N, 1), _f32)],
    )(acc, denp, b, w, a_s, a_d)


def _final_body(e1_ref, acc_ref, den_ref, b_ref, wd_ref, out_ref):
    r = 1.0 / (den_ref[:, 0:1] + den_ref[:, 1:2] + 1e-16)
    emb = e1_ref[...] + (acc_ref[0] + acc_ref[1]) * r + b_ref[...]
    out_ref[...] = jnp.dot(emb, wd_ref[...], preferred_element_type=_f32)


def _tc_final(e1, acc, denp, b, wd):
    return pl.pallas_call(
        _final_body,
        grid=(10,),
        in_specs=[pl.BlockSpec((1000, D), lambda i: (i, 0)),
                  pl.BlockSpec((2, 1000, D), lambda i: (0, i, 0)),
                  pl.BlockSpec((1000, D), lambda i: (i, 0)),
                  pl.BlockSpec((1, D), lambda i: (0, 0)),
                  pl.BlockSpec((D, D), lambda i: (0, 0))],
        out_specs=pl.BlockSpec((1000, D), lambda i: (i, 0)),
        out_shape=jax.ShapeDtypeStruct((N, D), _f32),
    )(e1, acc, denp, b, wd)


def _den_cols(den):
    d = den.reshape(2, DROWS * D)[:, :N]
    return jnp.pad(jnp.swapaxes(d, 0, 1), ((0, 0), (0, D - 2)))


def kernel(x, edge_index, batch, W1, a_src1, a_dst1, b1,
           W2, a_src2, a_dst2, b2, Wd):
    loops = jnp.arange(N, dtype=jnp.int32)
    src = jnp.concatenate([edge_index[0], loops])
    dst = jnp.concatenate([edge_index[1], loops])
    src_w = jnp.pad(src, (0, EPAD - E_TOT)).reshape(NW, EW)
    dst_w = jnp.pad(dst, (0, EPAD - E_TOT)).reshape(NW, EW)

    h1, as1, ad1 = _tc_proj(x, W1, a_src1.reshape(D, 1), a_dst1.reshape(D, 1))
    acc1, den1 = _sc_gat(h1, as1.reshape(N), ad1.reshape(N), src_w, dst_w)
    e1, h2, as2, ad2 = _tc_comb_proj(acc1, _den_cols(den1), b1.reshape(1, D),
                                     W2, a_src2.reshape(D, 1),
                                     a_dst2.reshape(D, 1))
    acc2, den2 = _sc_gat(h2, as2.reshape(N), ad2.reshape(N), src_w, dst_w)
    out = _tc_final(e1, acc2, _den_cols(den2), b2.reshape(1, D), Wd)
    return (out, batch)


# speedup vs baseline: 1.0393x; 1.0393x over previous
import functools

import jax
import jax.numpy as jnp
from jax import lax
from jax.experimental import pallas as pl
from jax.experimental.pallas import tpu as pltpu
from jax.experimental.pallas import tpu_sc as plsc

N = 10000
D = 128
E_TOT = 330000
NW = 32
K = 64
NB = 162
EW = NB * K
EPAD = NW * EW
NP = NB // 2
DROWS = 80

_f32 = jnp.float32


def _sc_gat_body(h_hbm, asrc_hbm, adst_hbm, srcf_hbm, dstf_hbm,
                 acc_out, den_out,
                 src_f, dst_f, denom_l, buf0, buf1, asg0, adg0, asg1, adg1,
                 acc_sh, den_sh, asrc_sp, adst_sp,
                 semr0, semr1, sema0, sema1, sems0, sems1):
    c = lax.axis_index("c")
    s = lax.axis_index("s")
    wid = s * 2 + c
    iota16 = lax.iota(jnp.int32, 16)
    zeros16 = jnp.zeros((16,), _f32)

    def in_copies():
        cps = [pltpu.make_async_copy(srcf_hbm.at[wid], src_f, semr0),
               pltpu.make_async_copy(dstf_hbm.at[wid], dst_f, semr0)]
        return cps

    def tbl_copies():
        return [pltpu.make_async_copy(asrc_hbm, asrc_sp, semr1),
                pltpu.make_async_copy(adst_hbm, adst_sp, semr1)]

    for cp in in_copies():
        cp.start()

    @pl.when(s == 0)
    def _():
        for cp in tbl_copies():
            cp.start()

    for r in range(DROWS):
        for cc in range(8):
            denom_l[r, pl.ds(16 * cc, 16)] = zeros16
    for r in range(K):
        for cc in range(8):
            buf0[r, pl.ds(16 * cc, 16)] = zeros16

    abase = pl.multiple_of(s * 632, 8)
    dbase = pl.multiple_of(s * 8, 8)

    def zero_copies_a():
        return [pltpu.make_async_copy(buf0, acc_sh.at[pl.ds(abase + 64 * r, 64)],
                                      sems0) for r in range(9)] + \
               [pltpu.make_async_copy(buf0.at[pl.ds(0, 56)],
                                      acc_sh.at[pl.ds(abase + 576, 56)], sems0)]

    def zero_copies_b():
        return [pltpu.make_async_copy(buf0, acc_sh.at[pl.ds(abase + 64 * r, 64)],
                                      sems0) for r in range(8)] + \
               [pltpu.make_async_copy(buf0.at[pl.ds(0, 8)],
                                      acc_sh.at[pl.ds(abase + 512, 8)], sems0)]

    def zero_copies_d():
        return [pltpu.make_async_copy(denom_l.at[pl.ds(0, 8)],
                                      den_sh.at[pl.ds(dbase, 8)], sems1)]

    @pl.when(s < 15)
    def _():
        for cp in zero_copies_a():
            cp.start()

    @pl.when(s == 15)
    def _():
        for cp in zero_copies_b():
            cp.start()

    @pl.when(s < 10)
    def _():
        for cp in zero_copies_d():
            cp.start()

    for cp in in_copies():
        cp.wait()

    @pl.when(s == 0)
    def _():
        for cp in tbl_copies():
            cp.wait()

    @pl.when(s < 15)
    def _():
        for cp in zero_copies_a():
            cp.wait()

    @pl.when(s == 15)
    def _():
        for cp in zero_copies_b():
            cp.wait()

    @pl.when(s < 10)
    def _():
        for cp in zero_copies_d():
            cp.wait()

    plsc.subcore_barrier()

    def copies(j, buf, asg, adg, semr, sema):
        sidx = src_f.at[pl.ds(j * K, K)]
        didx = dst_f.at[pl.ds(j * K, K)]
        return (pltpu.make_async_copy(h_hbm.at[sidx], buf, semr),
                pltpu.make_async_copy(asrc_sp.at[sidx], asg, sema),
                pltpu.make_async_copy(adst_sp.at[didx], adg, sema))

    def fire(j, buf, asg, adg, semr, sema):
        for cp in copies(j, buf, asg, adg, semr, sema):
            cp.start()

    def wait(j, buf, asg, adg, semr, sema):
        for cp in copies(j, buf, asg, adg, semr, sema):
            cp.wait()

    def compute(j, buf, asg, adg, sems):
        base = wid * EW + j * K
        dis = []
        for sg in range(4):
            sl16 = pl.ds(16 * sg, 16)
            di = dst_f[pl.ds(j * K + 16 * sg, 16)]
            e = asg[sl16] + adg[sl16]
            e = jnp.where(e >= 0.0, e, e * jnp.float32(0.2))
            ex = jnp.exp(e)
            ex = jnp.where(base + 16 * sg + iota16 < E_TOT, ex, 0.0)
            plsc.addupdate_scatter(denom_l, [di >> 7, di & 127], ex)
            for k in range(16):
                bc = jnp.full((16,), ex[k], _f32)
                row = 16 * sg + k
                for cc in range(8):
                    cs = pl.ds(16 * cc, 16)
                    buf[row, cs] = buf[row, cs] * bc
            dis.append(di)
        for sg in range(4):
            pltpu.make_async_copy(buf.at[pl.ds(16 * sg, 16)],
                                  acc_sh.at[dis[sg]], sems).start(add=True)

    def wait_scat(buf, sems):
        for sg in range(4):
            pltpu.make_async_copy(buf.at[pl.ds(16 * sg, 16)],
                                  acc_sh.at[iota16], sems).wait()

    fire(0, buf0, asg0, adg0, semr0, sema0)

    def body(jp, carry):
        j0 = 2 * jp
        j1 = j0 + 1

        @pl.when(jp > 0)
        def _():
            wait_scat(buf1, sems1)

        fire(j1, buf1, asg1, adg1, semr1, sema1)
        wait(j0, buf0, asg0, adg0, semr0, sema0)
        compute(j0, buf0, asg0, adg0, sems0)

        @pl.when(jp < NP - 1)
        def _():
            wait_scat(buf0, sems0)
            fire(j0 + 2, buf0, asg0, adg0, semr0, sema0)

        wait(j1, buf1, asg1, adg1, semr1, sema1)
        compute(j1, buf1, asg1, adg1, sems1)
        return carry

    lax.fori_loop(0, NP, body, 0)
    wait_scat(buf0, sems0)
    wait_scat(buf1, sems1)

    dcomb = [pltpu.make_async_copy(denom_l.at[pl.ds(16 * g, 16)],
                                   den_sh.at[iota16 + 16 * g], sems0)
             for g in range(5)]
    for cp in dcomb:
        cp.start(add=True)
    for cp in dcomb:
        cp.wait()
    plsc.subcore_barrier()

    @pl.when(s < 15)
    def _():
        cps = [pltpu.make_async_copy(acc_sh.at[pl.ds(abase, 632)],
                                     acc_out.at[c, pl.ds(abase, 632)], semr0)]
        for cp in cps:
            cp.start()

    @pl.when(s == 15)
    def _():
        cps = [pltpu.make_async_copy(acc_sh.at[pl.ds(abase, 520)],
                                     acc_out.at[c, pl.ds(abase, 520)], semr0)]
        for cp in cps:
            cp.start()

    @pl.when(s < 10)
    def _():
        pltpu.sync_copy(den_sh.at[pl.ds(dbase, 8)],
                        den_out.at[c, pl.ds(dbase, 8)])

    @pl.when(s < 15)
    def _():
        pltpu.make_async_copy(acc_sh.at[pl.ds(abase, 632)],
                              acc_out.at[c, pl.ds(abase, 632)], semr0).wait()

    @pl.when(s == 15)
    def _():
        pltpu.make_async_copy(acc_sh.at[pl.ds(abase, 520)],
                              acc_out.at[c, pl.ds(abase, 520)], semr0).wait()


_sc_gat = functools.partial(
    pl.kernel,
    out_type=(jax.ShapeDtypeStruct((2, N, D), _f32),
              jax.ShapeDtypeStruct((2, DROWS, D), _f32)),
    mesh=plsc.VectorSubcoreMesh(core_axis_name="c", subcore_axis_name="s"),
    compiler_params=pltpu.CompilerParams(needs_layout_passes=False),
    scratch_types=[
        pltpu.VMEM((EW,), jnp.int32),
        pltpu.VMEM((EW,), jnp.int32),
        pltpu.VMEM((DROWS, D), _f32),
        pltpu.VMEM((K, D), _f32),
        pltpu.VMEM((K, D), _f32),
        pltpu.VMEM((K,), _f32),
        pltpu.VMEM((K,), _f32),
        pltpu.VMEM((K,), _f32),
        pltpu.VMEM((K,), _f32),
        pltpu.VMEM_SHARED((N, D), _f32),
        pltpu.VMEM_SHARED((DROWS, D), _f32),
        pltpu.VMEM_SHARED((N,), _f32),
        pltpu.VMEM_SHARED((N,), _f32),
        pltpu.SemaphoreType.DMA,
        pltpu.SemaphoreType.DMA,
        pltpu.SemaphoreType.DMA,
        pltpu.SemaphoreType.DMA,
        pltpu.SemaphoreType.DMA,
        pltpu.SemaphoreType.DMA,
    ],
)(_sc_gat_body)


def _proj_body(x_ref, w_ref, a_ref, h_ref, sc_ref):
    h = jnp.dot(x_ref[...], w_ref[...], preferred_element_type=_f32)
    h_ref[...] = h
    sc_ref[...] = jnp.dot(h, a_ref[...], preferred_element_type=_f32)


def _tc_proj(x, w, a):
    return pl.pallas_call(
        _proj_body,
        grid=(10,),
        in_specs=[pl.BlockSpec((1000, D), lambda i: (i, 0)),
                  pl.BlockSpec((D, D), lambda i: (0, 0)),
                  pl.BlockSpec((D, D), lambda i: (0, 0))],
        out_specs=[pl.BlockSpec((1000, D), lambda i: (i, 0)),
                   pl.BlockSpec((1000, D), lambda i: (i, 0))],
        out_shape=[jax.ShapeDtypeStruct((N, D), _f32),
                   jax.ShapeDtypeStruct((N, D), _f32)],
    )(x, w, a)


def _comb_proj_body(acc_ref, den_ref, b_ref, w_ref, a_ref,
                    e_ref, h_ref, sc_ref):
    r = 1.0 / (den_ref[:, 0:1] + den_ref[:, 1:2] + 1e-16)
    e = (acc_ref[0] + acc_ref[1]) * r + b_ref[...]
    e_ref[...] = e
    h = jnp.dot(e, w_ref[...], preferred_element_type=_f32)
    h_ref[...] = h
    sc_ref[...] = jnp.dot(h, a_ref[...], preferred_element_type=_f32)


def _tc_comb_proj(acc, denp, b, w, a):
    return pl.pallas_call(
        _comb_proj_body,
        grid=(10,),
        in_specs=[pl.BlockSpec((2, 1000, D), lambda i: (0, i, 0)),
                  pl.BlockSpec((1000, D), lambda i: (i, 0)),
                  pl.BlockSpec((1, D), lambda i: (0, 0)),
                  pl.BlockSpec((D, D), lambda i: (0, 0)),
                  pl.BlockSpec((D, D), lambda i: (0, 0))],
        out_specs=[pl.BlockSpec((1000, D), lambda i: (i, 0)),
                   pl.BlockSpec((1000, D), lambda i: (i, 0)),
                   pl.BlockSpec((1000, D), lambda i: (i, 0))],
        out_shape=[jax.ShapeDtypeStruct((N, D), _f32),
                   jax.ShapeDtypeStruct((N, D), _f32),
                   jax.ShapeDtypeStruct((N, D), _f32)],
    )(acc, denp, b, w, a)


def _final_body(e1_ref, acc_ref, den_ref, b_ref, wd_ref, out_ref):
    r = 1.0 / (den_ref[:, 0:1] + den_ref[:, 1:2] + 1e-16)
    emb = e1_ref[...] + (acc_ref[0] + acc_ref[1]) * r + b_ref[...]
    out_ref[...] = jnp.dot(emb, wd_ref[...], preferred_element_type=_f32)


def _tc_final(e1, acc, denp, b, wd):
    return pl.pallas_call(
        _final_body,
        grid=(10,),
        in_specs=[pl.BlockSpec((1000, D), lambda i: (i, 0)),
                  pl.BlockSpec((2, 1000, D), lambda i: (0, i, 0)),
                  pl.BlockSpec((1000, D), lambda i: (i, 0)),
                  pl.BlockSpec((1, D), lambda i: (0, 0)),
                  pl.BlockSpec((D, D), lambda i: (0, 0))],
        out_specs=pl.BlockSpec((1000, D), lambda i: (i, 0)),
        out_shape=jax.ShapeDtypeStruct((N, D), _f32),
    )(e1, acc, denp, b, wd)


def _den_cols(den):
    d = den.reshape(2, DROWS * D)[:, :N]
    return jnp.pad(jnp.swapaxes(d, 0, 1), ((0, 0), (0, D - 2)))


def kernel(x, edge_index, batch, W1, a_src1, a_dst1, b1,
           W2, a_src2, a_dst2, b2, Wd):
    loops = jnp.arange(N, dtype=jnp.int32)
    src = jnp.concatenate([edge_index[0], loops])
    dst = jnp.concatenate([edge_index[1], loops])
    src_w = jnp.pad(src, (0, EPAD - E_TOT)).reshape(NW, EW)
    dst_w = jnp.pad(dst, (0, EPAD - E_TOT)).reshape(NW, EW)

    def attn_mat(a_s, a_d):
        a = jnp.zeros((D, D), _f32)
        return a.at[:, 0].set(a_s.reshape(-1)).at[:, 1].set(a_d.reshape(-1))

    h1, sc1 = _tc_proj(x, W1, attn_mat(a_src1, a_dst1))
    acc1, den1 = _sc_gat(h1, sc1[:, 0], sc1[:, 1], src_w, dst_w)
    e1, h2, sc2 = _tc_comb_proj(acc1, _den_cols(den1), b1.reshape(1, D),
                                W2, attn_mat(a_src2, a_dst2))
    acc2, den2 = _sc_gat(h2, sc2[:, 0], sc2[:, 1], src_w, dst_w)
    out = _tc_final(e1, acc2, _den_cols(den2), b2.reshape(1, D), Wd)
    return (out, batch)
